# baseline (speedup 1.0000x reference)
import functools

import jax
import jax.numpy as jnp
from jax import lax
from jax.experimental import pallas as pl
from jax.experimental.pallas import tpu as pltpu
from jax.experimental.pallas import tpu_sc as plsc

N_NODE = 50000
E = 320000
D_IN = 128
D_OUT = 64

NC = 2
NS = 16
L = 16

CHUNK = 64
BI = 16
NBI = 20
J = BI * NBI
EPT = J * CHUNK
E_PAD = NS * EPT

HALF = N_NODE // NC
ROWS_PT = 1568
H_ACC = NS * ROWS_PT
GARBAGE = 25080
CNT_W = 8
OUT_PT = 1568
PAD_DST = 1 << 29


def _mm_body(x_ref, w_ref, b_ref, o_ref):
    o_ref[...] = (
        jnp.dot(x_ref[...], w_ref[...], preferred_element_type=jnp.float32)
        + b_ref[...]
    )


@jax.jit
def _linear(x, W, b):
    blk = 1000
    return pl.pallas_call(
        _mm_body,
        grid=(N_NODE // blk,),
        in_specs=[
            pl.BlockSpec((blk, D_IN), lambda i: (i, 0)),
            pl.BlockSpec((D_IN, D_OUT), lambda i: (0, 0)),
            pl.BlockSpec((1, D_OUT), lambda i: (0, 0)),
        ],
        out_specs=pl.BlockSpec((blk, D_OUT), lambda i: (i, 0)),
        out_shape=jax.ShapeDtypeStruct((N_NODE, D_OUT), jnp.float32),
    )(x, W, b.reshape(1, D_OUT))


def _sc_body(wh_hbm, src_hbm, dst_hbm, ones_hbm, zeros_hbm,
             sum_out, cnt_out,
             src_blk, dstl_blk, rows_a, rows_b, ones_v,
             acc_sh, cnt_sh, sem_a, sem_b):
    c = lax.axis_index("c")
    s = lax.axis_index("s")

    zv = jnp.zeros((L,), jnp.float32)

    @pl.loop(0, CHUNK)
    def _(r):
        for k in range(D_OUT // L):
            rows_a[r, pl.ds(k * L, L)] = zv

    pltpu.sync_copy(ones_hbm, ones_v)

    a0 = s * ROWS_PT
    for k in range(ROWS_PT // CHUNK):
        pltpu.sync_copy(rows_a, acc_sh.at[pl.ds(a0 + k * CHUNK, CHUNK)])
        pltpu.sync_copy(zeros_hbm, cnt_sh.at[pl.ds(a0 + k * CHUNK, CHUNK)])
    tail = ROWS_PT % CHUNK
    if tail:
        t0 = a0 + ROWS_PT - tail
        pltpu.sync_copy(rows_a.at[pl.ds(0, tail)], acc_sh.at[pl.ds(t0, tail)])
        pltpu.sync_copy(zeros_hbm.at[pl.ds(0, tail)],
                        cnt_sh.at[pl.ds(t0, tail)])

    plsc.subcore_barrier()

    base = jnp.full((L,), c * HALF, jnp.int32)
    garbage = jnp.full((L,), GARBAGE, jnp.int32)
    bufs = (rows_a, rows_b)
    sems = (sem_a, sem_b)

    @pl.loop(0, NBI)
    def _(bi):
        pltpu.sync_copy(src_hbm.at[s, pl.ds(bi * BI, BI)], src_blk)
        pltpu.sync_copy(dst_hbm.at[s, pl.ds(bi * BI, BI)], dstl_blk)

        @pl.loop(0, BI)
        def _(r):
            for k in range(CHUNK // L):
                d = dstl_blk[r, pl.ds(k * L, L)]
                loc = d - base
                ok = (loc >= 0) & (loc < HALF)
                dstl_blk[r, pl.ds(k * L, L)] = jnp.where(ok, loc, garbage)

        descs = [None, None]
        descs[0] = pltpu.async_copy(wh_hbm.at[src_blk.at[0]], bufs[0], sems[0])
        for j in range(BI):
            b = j % 2
            descs[b].wait()
            if j + 1 < BI:
                nb = 1 - b
                descs[nb] = pltpu.async_copy(
                    wh_hbm.at[src_blk.at[j + 1]], bufs[nb], sems[nb])
            pltpu.sync_copy(bufs[b], acc_sh.at[dstl_blk.at[j]], add=True)
            pltpu.sync_copy(ones_v, cnt_sh.at[dstl_blk.at[j]], add=True)

    plsc.subcore_barrier()

    o0 = c * HALF

    @pl.when(s < NS - 1)
    def _():
        pltpu.sync_copy(acc_sh.at[pl.ds(s * OUT_PT, OUT_PT)],
                        sum_out.at[pl.ds(o0 + s * OUT_PT, OUT_PT)])
        pltpu.sync_copy(cnt_sh.at[pl.ds(s * OUT_PT, OUT_PT)],
                        cnt_out.at[pl.ds(o0 + s * OUT_PT, OUT_PT)])

    @pl.when(s == NS - 1)
    def _():
        tail0 = (NS - 1) * OUT_PT
        tail_n = HALF - tail0
        pltpu.sync_copy(acc_sh.at[pl.ds(tail0, tail_n)],
                        sum_out.at[pl.ds(o0 + tail0, tail_n)])
        pltpu.sync_copy(cnt_sh.at[pl.ds(tail0, tail_n)],
                        cnt_out.at[pl.ds(o0 + tail0, tail_n)])


_sc_aggregate = pl.kernel(
    _sc_body,
    out_type=[
        jax.ShapeDtypeStruct((N_NODE, D_OUT), jnp.float32),
        jax.ShapeDtypeStruct((N_NODE, CNT_W), jnp.float32),
    ],
    mesh=plsc.VectorSubcoreMesh(
        core_axis_name="c", subcore_axis_name="s",
        num_cores=NC, num_subcores=NS,
    ),
    compiler_params=pltpu.CompilerParams(use_tc_tiling_on_sc=False),
    scratch_types=[
        pltpu.VMEM((BI, CHUNK), jnp.int32),
        pltpu.VMEM((BI, CHUNK), jnp.int32),
        pltpu.VMEM((CHUNK, D_OUT), jnp.float32),
        pltpu.VMEM((CHUNK, D_OUT), jnp.float32),
        pltpu.VMEM((CHUNK, CNT_W), jnp.float32),
        pltpu.VMEM_SHARED((H_ACC, D_OUT), jnp.float32),
        pltpu.VMEM_SHARED((H_ACC, CNT_W), jnp.float32),
        pltpu.SemaphoreType.DMA,
        pltpu.SemaphoreType.DMA,
    ],
)


def _div_body(s_ref, c_ref, o_ref):
    cnt = jnp.maximum(c_ref[:, 0:1], 1.0)
    o_ref[...] = s_ref[...] / cnt


@jax.jit
def _mean_div(summed, counts):
    blk = 1000
    return pl.pallas_call(
        _div_body,
        grid=(N_NODE // blk,),
        in_specs=[
            pl.BlockSpec((blk, D_OUT), lambda i: (i, 0)),
            pl.BlockSpec((blk, CNT_W), lambda i: (i, 0)),
        ],
        out_specs=pl.BlockSpec((blk, D_OUT), lambda i: (i, 0)),
        out_shape=jax.ShapeDtypeStruct((N_NODE, D_OUT), jnp.float32),
    )(summed, counts)


def _pad_edges(edge_index):
    src = edge_index[0].astype(jnp.int32)
    dst = edge_index[1].astype(jnp.int32)
    pad = E_PAD - E
    src = jnp.concatenate([src, jnp.zeros((pad,), jnp.int32)])
    dst = jnp.concatenate([dst, jnp.full((pad,), PAD_DST, jnp.int32)])
    return src.reshape(NS, J, CHUNK), dst.reshape(NS, J, CHUNK)


def kernel(x_user, x_item, edge_index_e0, edge_index_e1, W_e0, b_e0, W_e1, b_e1):
    ones8 = jnp.ones((CHUNK, CNT_W), jnp.float32)
    zeros8 = jnp.zeros((CHUNK, CNT_W), jnp.float32)

    Wh_user = _linear(x_user, W_e0, b_e0)
    Wh_item = _linear(x_item, W_e1, b_e1)

    src0, dst0 = _pad_edges(edge_index_e0)
    src1, dst1 = _pad_edges(edge_index_e1)

    sum_item, cnt_item = _sc_aggregate(Wh_user, src0, dst0, ones8, zeros8)
    sum_user, cnt_user = _sc_aggregate(Wh_item, src1, dst1, ones8, zeros8)

    h_item = _mean_div(sum_item, cnt_item)
    h_user = _mean_div(sum_user, cnt_user)
    return (h_user, h_item)


# speedup vs baseline: 1.0750x; 1.0750x over previous
import functools

import jax
import jax.numpy as jnp
from jax import lax
from jax.experimental import pallas as pl
from jax.experimental.pallas import tpu as pltpu
from jax.experimental.pallas import tpu_sc as plsc

N_NODE = 50000
E = 320000
D_IN = 128
D_OUT = 64

NC = 2
NS = 16
L = 16

CHUNK = 64
BI = 8
NBI = 40
J = BI * NBI
EPT = J * CHUNK
E_PAD = NS * EPT
NBUF = 4

HALF = N_NODE // NC
ROWS_PT = 1563
H_ACC = NS * ROWS_PT
GARBAGE = 25000
CNT_W = 8
OUT_PT = 1568
PAD_DST = 1 << 29


def _mm_body(x_ref, w_ref, b_ref, o_ref):
    o_ref[...] = (
        jnp.dot(x_ref[...], w_ref[...], preferred_element_type=jnp.float32)
        + b_ref[...]
    )


@jax.jit
def _linear(x, W, b):
    blk = 1000
    return pl.pallas_call(
        _mm_body,
        grid=(N_NODE // blk,),
        in_specs=[
            pl.BlockSpec((blk, D_IN), lambda i: (i, 0)),
            pl.BlockSpec((D_IN, D_OUT), lambda i: (0, 0)),
            pl.BlockSpec((1, D_OUT), lambda i: (0, 0)),
        ],
        out_specs=pl.BlockSpec((blk, D_OUT), lambda i: (i, 0)),
        out_shape=jax.ShapeDtypeStruct((N_NODE, D_OUT), jnp.float32),
    )(x, W, b.reshape(1, D_OUT))


def _sc_body(wh_hbm, src_hbm, dst_hbm, ones_hbm, zeros_hbm,
             sum_out, cnt_out,
             src_blk, dst_a, dst_b, rows0, rows1, rows2, rows3, ones_v,
             acc_sh, cnt_sh,
             sg0, sg1, sg2, sg3, ss0, ss1, ss2, ss3, sem_c):
    c = lax.axis_index("c")
    s = lax.axis_index("s")
    bufs = (rows0, rows1, rows2, rows3)
    gsem = (sg0, sg1, sg2, sg3)
    ssem = (ss0, ss1, ss2, ss3)

    zv = jnp.zeros((L,), jnp.float32)

    @pl.loop(0, CHUNK)
    def _(r):
        for k in range(D_OUT // L):
            rows0[r, pl.ds(k * L, L)] = zv

    pltpu.sync_copy(ones_hbm, ones_v)

    a0 = s * ROWS_PT
    for k in range(ROWS_PT // CHUNK):
        pltpu.sync_copy(rows0, acc_sh.at[pl.ds(a0 + k * CHUNK, CHUNK)])
        pltpu.sync_copy(zeros_hbm, cnt_sh.at[pl.ds(a0 + k * CHUNK, CHUNK)])
    tail = ROWS_PT % CHUNK
    if tail:
        t0 = a0 + ROWS_PT - tail
        pltpu.sync_copy(rows0.at[pl.ds(0, tail)], acc_sh.at[pl.ds(t0, tail)])
        pltpu.sync_copy(zeros_hbm.at[pl.ds(0, tail)],
                        cnt_sh.at[pl.ds(t0, tail)])

    plsc.subcore_barrier()

    base = jnp.full((L,), c * HALF, jnp.int32)
    garbage = jnp.full((L,), GARBAGE, jnp.int32)

    def wait_scatter(b):
        pltpu.make_async_copy(bufs[b], acc_sh.at[pl.ds(0, CHUNK)],
                              ssem[b]).wait()

    def drain_counts(nchunks):
        n = nchunks * CHUNK
        pltpu.make_async_copy(cnt_out.at[pl.ds(0, n)],
                              cnt_sh.at[pl.ds(0, n)], sem_c).wait()

    def do_block(bi, dst_v, first, drain):
        if drain:
            drain_counts(BI)
        pltpu.sync_copy(src_hbm.at[s, pl.ds(bi * BI, BI)], src_blk)
        pltpu.sync_copy(dst_hbm.at[s, pl.ds(bi * BI, BI)], dst_v)

        for r in range(BI):
            for k in range(CHUNK // L):
                d = dst_v[r, pl.ds(k * L, L)]
                loc = d - base
                ok = (loc >= 0) & (loc < HALF)
                dst_v[r, pl.ds(k * L, L)] = jnp.where(ok, loc, garbage)

        for k in (0, 1):
            if not first:
                wait_scatter(k)
            pltpu.async_copy(wh_hbm.at[src_blk.at[k]], bufs[k], gsem[k])
        for k in range(BI):
            b = k % NBUF
            pltpu.make_async_copy(wh_hbm.at[src_blk.at[k]], bufs[b],
                                  gsem[b]).wait()
            pltpu.async_copy(bufs[b], acc_sh.at[dst_v.at[k]], ssem[b],
                             add=True)
            pltpu.async_copy(ones_v, cnt_sh.at[dst_v.at[k]], sem_c,
                             add=True)
            if k + 2 < BI:
                b2 = (k + 2) % NBUF
                if not (first and k < 2):
                    wait_scatter(b2)
                pltpu.async_copy(wh_hbm.at[src_blk.at[k + 2]], bufs[b2],
                                 gsem[b2])

    do_block(0, dst_a, True, False)
    do_block(1, dst_b, False, False)

    @pl.loop(2, NBI, step=2)
    def _(bi):
        do_block(bi, dst_a, False, True)
        do_block(bi + 1, dst_b, False, True)

    for b in range(NBUF):
        wait_scatter(b)
    drain_counts(2 * BI)

    plsc.subcore_barrier()

    o0 = c * HALF

    @pl.when(s < NS - 1)
    def _():
        pltpu.sync_copy(acc_sh.at[pl.ds(s * OUT_PT, OUT_PT)],
                        sum_out.at[pl.ds(o0 + s * OUT_PT, OUT_PT)])
        pltpu.sync_copy(cnt_sh.at[pl.ds(s * OUT_PT, OUT_PT)],
                        cnt_out.at[pl.ds(o0 + s * OUT_PT, OUT_PT)])

    @pl.when(s == NS - 1)
    def _():
        tail0 = (NS - 1) * OUT_PT
        tail_n = HALF - tail0
        pltpu.sync_copy(acc_sh.at[pl.ds(tail0, tail_n)],
                        sum_out.at[pl.ds(o0 + tail0, tail_n)])
        pltpu.sync_copy(cnt_sh.at[pl.ds(tail0, tail_n)],
                        cnt_out.at[pl.ds(o0 + tail0, tail_n)])


_sc_aggregate = pl.kernel(
    _sc_body,
    out_type=[
        jax.ShapeDtypeStruct((N_NODE, D_OUT), jnp.float32),
        jax.ShapeDtypeStruct((N_NODE, CNT_W), jnp.float32),
    ],
    mesh=plsc.VectorSubcoreMesh(
        core_axis_name="c", subcore_axis_name="s",
        num_cores=NC, num_subcores=NS,
    ),
    compiler_params=pltpu.CompilerParams(use_tc_tiling_on_sc=False),
    scratch_types=[
        pltpu.VMEM((BI, CHUNK), jnp.int32),
        pltpu.VMEM((BI, CHUNK), jnp.int32),
        pltpu.VMEM((BI, CHUNK), jnp.int32),
        pltpu.VMEM((CHUNK, D_OUT), jnp.float32),
        pltpu.VMEM((CHUNK, D_OUT), jnp.float32),
        pltpu.VMEM((CHUNK, D_OUT), jnp.float32),
        pltpu.VMEM((CHUNK, D_OUT), jnp.float32),
        pltpu.VMEM((CHUNK, CNT_W), jnp.float32),
        pltpu.VMEM_SHARED((H_ACC, D_OUT), jnp.float32),
        pltpu.VMEM_SHARED((H_ACC, CNT_W), jnp.float32),
        pltpu.SemaphoreType.DMA,
        pltpu.SemaphoreType.DMA,
        pltpu.SemaphoreType.DMA,
        pltpu.SemaphoreType.DMA,
        pltpu.SemaphoreType.DMA,
        pltpu.SemaphoreType.DMA,
        pltpu.SemaphoreType.DMA,
        pltpu.SemaphoreType.DMA,
        pltpu.SemaphoreType.DMA,
    ],
)


def _div_body(s_ref, c_ref, o_ref):
    cnt = jnp.maximum(c_ref[:, 0:1], 1.0)
    o_ref[...] = s_ref[...] / cnt


@jax.jit
def _mean_div(summed, counts):
    blk = 1000
    return pl.pallas_call(
        _div_body,
        grid=(N_NODE // blk,),
        in_specs=[
            pl.BlockSpec((blk, D_OUT), lambda i: (i, 0)),
            pl.BlockSpec((blk, CNT_W), lambda i: (i, 0)),
        ],
        out_specs=pl.BlockSpec((blk, D_OUT), lambda i: (i, 0)),
        out_shape=jax.ShapeDtypeStruct((N_NODE, D_OUT), jnp.float32),
    )(summed, counts)


def _pad_edges(edge_index):
    src = edge_index[0].astype(jnp.int32)
    dst = edge_index[1].astype(jnp.int32)
    pad = E_PAD - E
    src = jnp.concatenate([src, jnp.zeros((pad,), jnp.int32)])
    dst = jnp.concatenate([dst, jnp.full((pad,), PAD_DST, jnp.int32)])
    return src.reshape(NS, J, CHUNK), dst.reshape(NS, J, CHUNK)


def kernel(x_user, x_item, edge_index_e0, edge_index_e1, W_e0, b_e0, W_e1, b_e1):
    ones8 = jnp.ones((CHUNK, CNT_W), jnp.float32)
    zeros8 = jnp.zeros((CHUNK, CNT_W), jnp.float32)

    Wh_user = _linear(x_user, W_e0, b_e0)
    Wh_item = _linear(x_item, W_e1, b_e1)

    src0, dst0 = _pad_edges(edge_index_e0)
    src1, dst1 = _pad_edges(edge_index_e1)

    sum_item, cnt_item = _sc_aggregate(Wh_user, src0, dst0, ones8, zeros8)
    sum_user, cnt_user = _sc_aggregate(Wh_item, src1, dst1, ones8, zeros8)

    h_item = _mean_div(sum_item, cnt_item)
    h_user = _mean_div(sum_user, cnt_user)
    return (h_user, h_item)


# speedup vs baseline: 1.3391x; 1.2456x over previous
import functools

import jax
import jax.numpy as jnp
from jax import lax
from jax.experimental import pallas as pl
from jax.experimental.pallas import tpu as pltpu
from jax.experimental.pallas import tpu_sc as plsc

N_NODE = 50000
E = 320000
D_IN = 128
D_OUT = 64

NC = 2
NS = 16
L = 16

CHUNK = 64
BI = 8
NBI = 40
J = BI * NBI
EPT = J * CHUNK
E_PAD = NS * EPT
NBUF = 4
NCH_CAP = 336

HALF = N_NODE // NC
ROWS_PT = 1563
H_ACC = NS * ROWS_PT
GARBAGE = 25000
CNT_W = 8
OUT_PT = 1568
PAD_DST = 1 << 29


def _mm_body(x_ref, w_ref, b_ref, o_ref):
    o_ref[...] = (
        jnp.dot(x_ref[...], w_ref[...], preferred_element_type=jnp.float32)
        + b_ref[...]
    )


@jax.jit
def _linear(x, W, b):
    blk = 1000
    return pl.pallas_call(
        _mm_body,
        grid=(N_NODE // blk,),
        in_specs=[
            pl.BlockSpec((blk, D_IN), lambda i: (i, 0)),
            pl.BlockSpec((D_IN, D_OUT), lambda i: (0, 0)),
            pl.BlockSpec((1, D_OUT), lambda i: (0, 0)),
        ],
        out_specs=pl.BlockSpec((blk, D_OUT), lambda i: (i, 0)),
        out_shape=jax.ShapeDtypeStruct((N_NODE, D_OUT), jnp.float32),
    )(x, W, b.reshape(1, D_OUT))


def _sc_body(wh_hbm, src_hbm, dst_hbm, ones_hbm, zeros_hbm,
             sum_out, cnt_out, csrc, cdst,
             src_blk, dst_blk, cb_src, cb_dst, fb_src, fb_dst,
             rows0, rows1, rows2, rows3, ones_v,
             acc_sh, cnt_sh,
             sg0, sg1, sg2, sg3, ss0, ss1, ss2, ss3, sem_c, fsem):
    c = lax.axis_index("c")
    s = lax.axis_index("s")
    cw = c * NS + s
    bufs = (rows0, rows1, rows2, rows3)
    gsem = (sg0, sg1, sg2, sg3)
    ssem = (ss0, ss1, ss2, ss3)

    zv = jnp.zeros((L,), jnp.float32)

    @pl.loop(0, CHUNK)
    def _(r):
        for k in range(D_OUT // L):
            rows0[r, pl.ds(k * L, L)] = zv

    pltpu.sync_copy(ones_hbm, ones_v)

    a0 = s * ROWS_PT
    for k in range(ROWS_PT // CHUNK):
        pltpu.sync_copy(rows0, acc_sh.at[pl.ds(a0 + k * CHUNK, CHUNK)])
        pltpu.sync_copy(zeros_hbm, cnt_sh.at[pl.ds(a0 + k * CHUNK, CHUNK)])
    tail = ROWS_PT % CHUNK
    if tail:
        t0 = a0 + ROWS_PT - tail
        pltpu.sync_copy(rows0.at[pl.ds(0, tail)], acc_sh.at[pl.ds(t0, tail)])
        pltpu.sync_copy(zeros_hbm.at[pl.ds(0, tail)],
                        cnt_sh.at[pl.ds(t0, tail)])

    plsc.subcore_barrier()

    base = jnp.full((L,), c * HALF, jnp.int32)
    garbage = jnp.full((L,), GARBAGE, jnp.int32)
    zeroi = jnp.zeros((L,), jnp.int32)
    lane = lax.iota(jnp.int32, L)

    def wait_flush():
        pltpu.make_async_copy(fb_src, csrc.at[cw, 0], fsem).wait()
        pltpu.make_async_copy(fb_dst, cdst.at[cw, 0], fsem).wait()

    pltpu.async_copy(fb_src, csrc.at[cw, NCH_CAP - 1], fsem)
    pltpu.async_copy(fb_dst, cdst.at[cw, NCH_CAP - 1], fsem)

    @pl.loop(0, NBI, init_carry=(jnp.int32(0), jnp.int32(0)))
    def p1(bi, carry):
        fill, wch = carry
        pltpu.sync_copy(src_hbm.at[s, pl.ds(bi * BI, BI)], src_blk)
        pltpu.sync_copy(dst_hbm.at[s, pl.ds(bi * BI, BI)], dst_blk)
        for k in range(BI):
            for v in range(CHUNK // L):
                d = dst_blk[k, pl.ds(v * L, L)]
                sv = src_blk[k, pl.ds(v * L, L)]
                loc = d - base
                ok = (loc >= 0) & (loc < HALF)
                oki = ok.astype(jnp.int32)
                csum = plsc.cumsum(oki)
                pos = jnp.full((L,), fill, jnp.int32) + csum - oki
                plsc.store_scatter(cb_src, [pos], sv, mask=ok)
                plsc.store_scatter(cb_dst, [pos], loc, mask=ok)
                fill = fill + plsc.all_reduce_population_count(ok)[0]
                flush = fill >= CHUNK

                @pl.when(flush)
                def _():
                    wait_flush()
                    for v2 in range(CHUNK // L):
                        fb_src[pl.ds(v2 * L, L)] = cb_src[pl.ds(v2 * L, L)]
                        fb_dst[pl.ds(v2 * L, L)] = cb_dst[pl.ds(v2 * L, L)]
                    pltpu.async_copy(fb_src, csrc.at[cw, wch], fsem)
                    pltpu.async_copy(fb_dst, cdst.at[cw, wch], fsem)
                    cb_src[pl.ds(0, L)] = cb_src[pl.ds(CHUNK, L)]
                    cb_dst[pl.ds(0, L)] = cb_dst[pl.ds(CHUNK, L)]

                fill = jnp.where(flush, fill - CHUNK, fill)
                wch = jnp.where(flush, wch + 1, wch)
        return fill, wch

    fill, wch = p1

    for v in range(CHUNK // L):
        keep = (lane + v * L) < fill
        cb_src[pl.ds(v * L, L)] = jnp.where(keep, cb_src[pl.ds(v * L, L)],
                                            zeroi)
        cb_dst[pl.ds(v * L, L)] = jnp.where(keep, cb_dst[pl.ds(v * L, L)],
                                            garbage)
    wait_flush()
    pltpu.sync_copy(cb_src.at[pl.ds(0, CHUNK)], csrc.at[cw, wch])
    pltpu.sync_copy(cb_dst.at[pl.ds(0, CHUNK)], cdst.at[cw, wch])
    nch = wch + 1

    for v in range(CHUNK // L):
        fb_src[pl.ds(v * L, L)] = zeroi
        fb_dst[pl.ds(v * L, L)] = garbage
    nch8 = ((nch + BI - 1) // BI) * BI

    @pl.loop(nch, nch8)
    def _(ch):
        pltpu.sync_copy(fb_src, csrc.at[cw, ch])
        pltpu.sync_copy(fb_dst, cdst.at[cw, ch])

    def wait_scatter(b):
        pltpu.make_async_copy(bufs[b], acc_sh.at[pl.ds(0, CHUNK)],
                              ssem[b]).wait()

    @pl.loop(0, nch8 // BI)
    def _(bi):
        pltpu.sync_copy(csrc.at[cw, pl.ds(bi * BI, BI)], src_blk)
        pltpu.sync_copy(cdst.at[cw, pl.ds(bi * BI, BI)], dst_blk)
        pltpu.async_copy(wh_hbm.at[src_blk.at[0]], bufs[0], gsem[0])
        pltpu.async_copy(wh_hbm.at[src_blk.at[1]], bufs[1], gsem[1])
        for k in range(BI):
            b = k % NBUF
            pltpu.make_async_copy(wh_hbm.at[src_blk.at[k]], bufs[b],
                                  gsem[b]).wait()
            pltpu.async_copy(bufs[b], acc_sh.at[dst_blk.at[k]], ssem[b],
                             add=True)
            pltpu.async_copy(ones_v, cnt_sh.at[dst_blk.at[k]], sem_c,
                             add=True)
            if k + 2 < BI:
                b2 = (k + 2) % NBUF
                if k >= 2:
                    wait_scatter(b2)
                pltpu.async_copy(wh_hbm.at[src_blk.at[k + 2]], bufs[b2],
                                 gsem[b2])
        for b in range(NBUF):
            wait_scatter(b)
        pltpu.make_async_copy(cnt_out.at[pl.ds(0, BI * CHUNK)],
                              cnt_sh.at[pl.ds(0, BI * CHUNK)], sem_c).wait()

    plsc.subcore_barrier()

    o0 = c * HALF

    @pl.when(s < NS - 1)
    def _():
        pltpu.sync_copy(acc_sh.at[pl.ds(s * OUT_PT, OUT_PT)],
                        sum_out.at[pl.ds(o0 + s * OUT_PT, OUT_PT)])
        pltpu.sync_copy(cnt_sh.at[pl.ds(s * OUT_PT, OUT_PT)],
                        cnt_out.at[pl.ds(o0 + s * OUT_PT, OUT_PT)])

    @pl.when(s == NS - 1)
    def _():
        tail0 = (NS - 1) * OUT_PT
        tail_n = HALF - tail0
        pltpu.sync_copy(acc_sh.at[pl.ds(tail0, tail_n)],
                        sum_out.at[pl.ds(o0 + tail0, tail_n)])
        pltpu.sync_copy(cnt_sh.at[pl.ds(tail0, tail_n)],
                        cnt_out.at[pl.ds(o0 + tail0, tail_n)])


_sc_aggregate = pl.kernel(
    _sc_body,
    out_type=[
        jax.ShapeDtypeStruct((N_NODE, D_OUT), jnp.float32),
        jax.ShapeDtypeStruct((N_NODE, CNT_W), jnp.float32),
        jax.ShapeDtypeStruct((NC * NS, NCH_CAP, CHUNK), jnp.int32),
        jax.ShapeDtypeStruct((NC * NS, NCH_CAP, CHUNK), jnp.int32),
    ],
    mesh=plsc.VectorSubcoreMesh(
        core_axis_name="c", subcore_axis_name="s",
        num_cores=NC, num_subcores=NS,
    ),
    compiler_params=pltpu.CompilerParams(use_tc_tiling_on_sc=False,
                                        needs_layout_passes=False),
    scratch_types=[
        pltpu.VMEM((BI, CHUNK), jnp.int32),
        pltpu.VMEM((BI, CHUNK), jnp.int32),
        pltpu.VMEM((CHUNK + L, ), jnp.int32),
        pltpu.VMEM((CHUNK + L, ), jnp.int32),
        pltpu.VMEM((CHUNK, ), jnp.int32),
        pltpu.VMEM((CHUNK, ), jnp.int32),
        pltpu.VMEM((CHUNK, D_OUT), jnp.float32),
        pltpu.VMEM((CHUNK, D_OUT), jnp.float32),
        pltpu.VMEM((CHUNK, D_OUT), jnp.float32),
        pltpu.VMEM((CHUNK, D_OUT), jnp.float32),
        pltpu.VMEM((CHUNK, CNT_W), jnp.float32),
        pltpu.VMEM_SHARED((H_ACC, D_OUT), jnp.float32),
        pltpu.VMEM_SHARED((H_ACC, CNT_W), jnp.float32),
        pltpu.SemaphoreType.DMA,
        pltpu.SemaphoreType.DMA,
        pltpu.SemaphoreType.DMA,
        pltpu.SemaphoreType.DMA,
        pltpu.SemaphoreType.DMA,
        pltpu.SemaphoreType.DMA,
        pltpu.SemaphoreType.DMA,
        pltpu.SemaphoreType.DMA,
        pltpu.SemaphoreType.DMA,
        pltpu.SemaphoreType.DMA,
    ],
)


def _div_body(s_ref, c_ref, o_ref):
    cnt = jnp.maximum(c_ref[:, 0:1], 1.0)
    o_ref[...] = s_ref[...] / cnt


@jax.jit
def _mean_div(summed, counts):
    blk = 1000
    return pl.pallas_call(
        _div_body,
        grid=(N_NODE // blk,),
        in_specs=[
            pl.BlockSpec((blk, D_OUT), lambda i: (i, 0)),
            pl.BlockSpec((blk, CNT_W), lambda i: (i, 0)),
        ],
        out_specs=pl.BlockSpec((blk, D_OUT), lambda i: (i, 0)),
        out_shape=jax.ShapeDtypeStruct((N_NODE, D_OUT), jnp.float32),
    )(summed, counts)


def _pad_edges(edge_index):
    src = edge_index[0].astype(jnp.int32)
    dst = edge_index[1].astype(jnp.int32)
    pad = E_PAD - E
    src = jnp.concatenate([src, jnp.zeros((pad,), jnp.int32)])
    dst = jnp.concatenate([dst, jnp.full((pad,), PAD_DST, jnp.int32)])
    return src.reshape(NS, J, CHUNK), dst.reshape(NS, J, CHUNK)


def kernel(x_user, x_item, edge_index_e0, edge_index_e1, W_e0, b_e0, W_e1, b_e1):
    ones8 = jnp.ones((CHUNK, CNT_W), jnp.float32)
    zeros8 = jnp.zeros((CHUNK, CNT_W), jnp.float32)

    Wh_user = _linear(x_user, W_e0, b_e0)
    Wh_item = _linear(x_item, W_e1, b_e1)

    src0, dst0 = _pad_edges(edge_index_e0)
    src1, dst1 = _pad_edges(edge_index_e1)

    sum_item, cnt_item, _, _ = _sc_aggregate(Wh_user, src0, dst0, ones8, zeros8)
    sum_user, cnt_user, _, _ = _sc_aggregate(Wh_item, src1, dst1, ones8, zeros8)

    h_item = _mean_div(sum_item, cnt_item)
    h_user = _mean_div(sum_user, cnt_user)
    return (h_user, h_item)
